# Initial kernel scaffold; baseline (speedup 1.0000x reference)
import jax
import jax.numpy as jnp
from jax.experimental import pallas as pl


def kernel(edge_index, adj_values, user_embedding, item_embedding):
    raise NotImplementedError("write your pallas kernel here")


# speedup vs baseline: 12.8842x; 12.8842x over previous
import functools

import jax
import jax.numpy as jnp
from jax import lax
from jax.experimental import pallas as pl
from jax.experimental.pallas import tpu as pltpu
from jax.experimental.pallas import tpu_sc as plsc

N_USERS = 30000
N_ITEMS = 20000
N = N_USERS + N_ITEMS
E = 1600000
D = 32
N_LAYERS = 3

LANES = 128
NW = 32
ROWS = 12544
E_PAD = ROWS * LANES
RPW = ROWS // NW
G_ROWS = 56
N_GROUPS = RPW // G_ROWS
N_PAD = 50176
TILE_ROWS = N_PAD // 16
ZCHUNK = 112
NZ = TILE_ROWS // ZCHUNK


def _layer_body(src_ref, dst_ref, vals_ref, x_ref, out_ref,
                acc, src_g, dst_g, vals_g, rows_v, sem):
    c = lax.axis_index("c")
    s = lax.axis_index("s")
    wid = s * 2 + c

    def zr(i, carry):
        rows_v[i, pl.ds(0, 16)] = jnp.zeros((16,), jnp.float32)
        rows_v[i, pl.ds(16, 16)] = jnp.zeros((16,), jnp.float32)
        return carry
    lax.fori_loop(0, LANES, zr, 0)

    zbase = s * TILE_ROWS

    def za(k, carry):
        pltpu.sync_copy(rows_v.at[pl.ds(0, ZCHUNK)],
                        acc.at[pl.ds(zbase + k * ZCHUNK, ZCHUNK)])
        return carry
    lax.fori_loop(0, NZ, za, 0)
    plsc.subcore_barrier()

    row_base = wid * RPW

    def group(gi, carry):
        gb = row_base + gi * G_ROWS
        pltpu.sync_copy(src_ref.at[pl.ds(gb, G_ROWS)], src_g)
        pltpu.sync_copy(dst_ref.at[pl.ds(gb, G_ROWS)], dst_g)
        pltpu.sync_copy(vals_ref.at[pl.ds(gb, G_ROWS)], vals_g)

        def chunk(jj, carry2):
            pltpu.async_copy(x_ref.at[src_g.at[jj]], rows_v, sem).wait()

            def scale(g16, carry3):
                gv = vals_g[jj, pl.ds(g16 * 16, 16)]
                for l in range(16):
                    i = g16 * 16 + l
                    g = gv[l]
                    rows_v[i, pl.ds(0, 16)] = rows_v[i, pl.ds(0, 16)] * g
                    rows_v[i, pl.ds(16, 16)] = rows_v[i, pl.ds(16, 16)] * g
                return carry3
            lax.fori_loop(0, LANES // 16, scale, 0)

            pltpu.sync_copy(rows_v, acc.at[dst_g.at[jj]], add=True)
            return carry2
        lax.fori_loop(0, G_ROWS, chunk, 0)
        return carry
    lax.fori_loop(0, N_GROUPS, group, 0)
    plsc.subcore_barrier()

    pltpu.sync_copy(acc.at[pl.ds(zbase, TILE_ROWS)],
                    out_ref.at[c, pl.ds(zbase, TILE_ROWS)])


@jax.jit
def _propagate(src2, dst2, vals2, x):
    mesh = plsc.VectorSubcoreMesh(core_axis_name="c", subcore_axis_name="s")
    layer = pl.kernel(
        _layer_body,
        mesh=mesh,
        compiler_params=pltpu.CompilerParams(use_tc_tiling_on_sc=False),
        out_type=jax.ShapeDtypeStruct((2, N_PAD, D), jnp.float32),
        scratch_types=[
            pltpu.VMEM_SHARED((N_PAD, D), jnp.float32),
            pltpu.VMEM((G_ROWS, LANES), jnp.int32),
            pltpu.VMEM((G_ROWS, LANES), jnp.int32),
            pltpu.VMEM((G_ROWS, LANES), jnp.float32),
            pltpu.VMEM((LANES, D), jnp.float32),
            pltpu.SemaphoreType.DMA,
        ],
    )
    acc = x
    for _ in range(N_LAYERS):
        p = layer(src2, dst2, vals2, x)
        x = (p[0] + p[1])[:N]
        acc = acc + x
    return acc * (1.0 / (N_LAYERS + 1))


def kernel(edge_index, adj_values, user_embedding, item_embedding):
    x = jnp.concatenate([user_embedding, item_embedding], axis=0)
    dst = edge_index[0].astype(jnp.int32)
    src = edge_index[1].astype(jnp.int32)
    vals = adj_values.astype(jnp.float32)
    npad = E_PAD - E
    pad_idx = (jnp.arange(npad, dtype=jnp.int32) * 37) % N
    src2 = jnp.concatenate([src, pad_idx]).reshape(ROWS, LANES)
    dst2 = jnp.concatenate([dst, pad_idx]).reshape(ROWS, LANES)
    vals2 = jnp.concatenate([vals, jnp.zeros((npad,), jnp.float32)]).reshape(ROWS, LANES)
    final = _propagate(src2, dst2, vals2, x)
    return (final[:N_USERS], final[N_USERS:])
